# baseline (speedup 1.0000x reference)
import functools

import jax
import jax.numpy as jnp
from jax.experimental import pallas as pl
from jax.experimental.pallas import tpu as pltpu


def _view_full(ei, ew, n):
    src = ei[0]
    dst = ei[1]
    if ew is None:
        ew = jnp.ones(src.shape, jnp.float32)
    ew = ew.astype(jnp.float32)
    vals = jnp.where(src == dst, ew - 1.0, ew)
    return jnp.eye(n, dtype=jnp.float32).at[dst, src].add(vals)


def _tile(n, target):
    if n % 8:
        return n
    best = 8
    for c in range(8, min(n, target) + 1, 8):
        if n % c == 0:
            best = c
    return best


def _deg_kernel(a_ref, deg_ref):
    deg_ref[...] = jnp.sum(a_ref[...], axis=1, keepdims=True)


def _xw_kernel(x_ref, w1_ref, xw_ref):
    xw_ref[...] = jnp.dot(
        x_ref[...].astype(jnp.bfloat16), w1_ref[...],
        preferred_element_type=jnp.float32).astype(jnp.bfloat16)


def _conv1_kernel(r1_ref, r2_ref, r3_ref, xw_ref, dr_ref, dc_ref,
                  b1_ref, w2_ref, feat_ref, z_ref, *, nhid):
    H = nhid
    feats = None
    zcols = []
    for v, r_ref in enumerate((r1_ref, r2_ref, r3_ref)):
        a = (r_ref[...] * dr_ref[v]).astype(jnp.bfloat16)
        m = jnp.dot(a, xw_ref[:, v * H:(v + 1) * H],
                    preferred_element_type=jnp.float32)
        h = jnp.maximum(dc_ref[v] * m + b1_ref[v], 0.0)
        feats = h if feats is None else feats + h
        if v != 1:
            k = 0 if v == 0 else 1
            zcols.append(jnp.sum(h * w2_ref[k], axis=1, keepdims=True))
    feat_ref[...] = feats
    z_ref[...] = jnp.concatenate(zcols, axis=1)


def _conv2_kernel(q1_ref, q3_ref, y_ref, dc_ref, b2c_ref, out_ref):
    s0 = jnp.sum(q1_ref[...] * y_ref[0:1, :], axis=1, keepdims=True)
    s1 = jnp.sum(q3_ref[...] * y_ref[1:2, :], axis=1, keepdims=True)
    out_ref[...] = dc_ref[:, 0:1] * s0 + dc_ref[:, 1:2] * s1 + b2c_ref[0, 0]


def kernel(x, ei1, ei2, ei3, ew1, ew2, ew3,
           w1_v1, b1_v1, w2_v1, b2_v1,
           w1_v2, b1_v2, w2_v2, b2_v2,
           w1_v3, b1_v3, w2_v3, b2_v3, att):
    N, F = x.shape
    H = w1_v1.shape[0]

    r1 = _view_full(ei1, ew1, N)
    r2 = _view_full(ei2, ew2, N)
    r3 = _view_full(ei3, ew3, N)

    TD = _tile(N, 128)
    deg_call = pl.pallas_call(
        _deg_kernel,
        out_shape=jax.ShapeDtypeStruct((N, 1), jnp.float32),
        grid=(N // TD,),
        in_specs=[pl.BlockSpec((TD, N), lambda i: (i, 0))],
        out_specs=pl.BlockSpec((TD, 1), lambda i: (i, 0)),
        compiler_params=pltpu.CompilerParams(
            dimension_semantics=("parallel",)),
    )
    dr1, dr2, dr3 = deg_call(r1), deg_call(r2), deg_call(r3)

    dep = 1e-38 * (dr1[0, 0] + dr2[0, 0] + dr3[0, 0])
    ones_dep1 = jnp.ones(ei1.shape[1], jnp.float32) + dep
    ones_dep3 = jnp.ones(ei3.shape[1], jnp.float32) + dep
    q1 = _view_full(ei1, ones_dep1, N)
    q3 = _view_full(ei3, ones_dep3, N)

    di1, di2, di3, qd1, qd3 = [
        jnp.where(d > 0.0, jax.lax.rsqrt(d), 0.0).reshape(N)
        for d in (dr1, dr2, dr3, deg_call(q1), deg_call(q3))]

    drow = jnp.stack([di1.reshape(1, N), di2.reshape(1, N),
                      di3.reshape(1, N)])
    dcol = jnp.stack([di1.reshape(N, 1), di2.reshape(N, 1),
                      di3.reshape(N, 1)])

    w1_all = jnp.concatenate(
        [w1_v1.T, w1_v2.T, w1_v3.T], axis=1).astype(jnp.bfloat16)
    b1_all = jnp.stack([b1_v1.reshape(1, H).astype(jnp.float32),
                        b1_v2.reshape(1, H).astype(jnp.float32),
                        b1_v3.reshape(1, H).astype(jnp.float32)])

    s = jax.nn.softmax(att.reshape(3).astype(jnp.float32))
    c0, c2 = s[0], s[1] + s[2]
    w2_rows = jnp.stack([c0 * w2_v1.reshape(1, H).astype(jnp.float32),
                         c2 * w2_v3.reshape(1, H).astype(jnp.float32)])
    b2c = (c0 * b2_v1.reshape(()).astype(jnp.float32)
           + c2 * b2_v3.reshape(()).astype(jnp.float32)).reshape(1, 1)

    TX = _tile(N, 512)
    xw = pl.pallas_call(
        _xw_kernel,
        out_shape=jax.ShapeDtypeStruct((N, 3 * H), jnp.bfloat16),
        grid=(N // TX,),
        in_specs=[pl.BlockSpec((TX, F), lambda i: (i, 0)),
                  pl.BlockSpec((F, 3 * H), lambda i: (0, 0))],
        out_specs=pl.BlockSpec((TX, 3 * H), lambda i: (i, 0)),
        compiler_params=pltpu.CompilerParams(
            dimension_semantics=("parallel",)),
    )(x, w1_all)

    T = _tile(N, 256)
    features, z = pl.pallas_call(
        functools.partial(_conv1_kernel, nhid=H),
        out_shape=(jax.ShapeDtypeStruct((N, H), x.dtype),
                   jax.ShapeDtypeStruct((N, 2), jnp.float32)),
        grid=(N // T,),
        in_specs=[pl.BlockSpec((T, N), lambda i: (i, 0)),
                  pl.BlockSpec((T, N), lambda i: (i, 0)),
                  pl.BlockSpec((T, N), lambda i: (i, 0)),
                  pl.BlockSpec((N, 3 * H), lambda i: (0, 0)),
                  pl.BlockSpec((3, 1, N), lambda i: (0, 0, 0)),
                  pl.BlockSpec((3, T, 1), lambda i: (0, i, 0)),
                  pl.BlockSpec((3, 1, H), lambda i: (0, 0, 0)),
                  pl.BlockSpec((2, 1, H), lambda i: (0, 0, 0))],
        out_specs=[pl.BlockSpec((T, H), lambda i: (i, 0)),
                   pl.BlockSpec((T, 2), lambda i: (i, 0))],
        compiler_params=pltpu.CompilerParams(
            dimension_semantics=("parallel",)),
    )(r1, r2, r3, xw, drow, dcol, b1_all, w2_rows)

    y = jnp.stack([qd1 * z[:, 0], qd3 * z[:, 1]])
    dc2 = jnp.stack([qd1, qd3], axis=1)
    TC = _tile(N, 256)
    comb = pl.pallas_call(
        _conv2_kernel,
        out_shape=jax.ShapeDtypeStruct((N, 1), x.dtype),
        grid=(N // TC,),
        in_specs=[pl.BlockSpec((TC, N), lambda i: (i, 0)),
                  pl.BlockSpec((TC, N), lambda i: (i, 0)),
                  pl.BlockSpec((2, N), lambda i: (0, 0)),
                  pl.BlockSpec((TC, 2), lambda i: (i, 0)),
                  pl.BlockSpec((1, 1), lambda i: (0, 0))],
        out_specs=pl.BlockSpec((TC, 1), lambda i: (i, 0)),
        compiler_params=pltpu.CompilerParams(
            dimension_semantics=("parallel",)),
    )(q1, q3, y, dc2, b2c)

    return comb.reshape(-1), features


# speedup vs baseline: 1.0711x; 1.0711x over previous
import functools

import jax
import jax.numpy as jnp
from jax.experimental import pallas as pl
from jax.experimental.pallas import tpu as pltpu


def _view_full(ei, ew, n):
    src = ei[0]
    dst = ei[1]
    if ew is None:
        ew = jnp.ones(src.shape, jnp.float32)
    ew = ew.astype(jnp.float32)
    vals = jnp.where(src == dst, ew - 1.0, ew)
    return jnp.eye(n, dtype=jnp.float32).at[dst, src].add(vals)


def _tile(n, target):
    if n % 8:
        return n
    best = 8
    for c in range(8, min(n, target) + 1, 8):
        if n % c == 0:
            best = c
    return best


def _deg_kernel(a_ref, deg_ref):
    deg_ref[...] = jnp.sum(a_ref[...], axis=1, keepdims=True)


def _xw_kernel(x_ref, w1_ref, xw_ref):
    xw_ref[...] = jnp.dot(
        x_ref[...].astype(jnp.bfloat16), w1_ref[...],
        preferred_element_type=jnp.float32).astype(jnp.bfloat16)


def _conv1_kernel(r1_ref, r2_ref, r3_ref, xw_ref, dr_ref, dc_ref,
                  b1_ref, w2_ref, feat_ref, z_ref, *, nhid):
    H = nhid
    feats = None
    zcols = []
    for v, r_ref in enumerate((r1_ref, r2_ref, r3_ref)):
        a = (r_ref[...] * dr_ref[v]).astype(jnp.bfloat16)
        m = jnp.dot(a, xw_ref[:, v * H:(v + 1) * H],
                    preferred_element_type=jnp.float32)
        h = jnp.maximum(dc_ref[v] * m + b1_ref[v], 0.0)
        feats = h if feats is None else feats + h
        if v != 1:
            k = 0 if v == 0 else 1
            zcols.append(jnp.sum(h * w2_ref[k], axis=1, keepdims=True))
    feat_ref[...] = feats
    z_ref[...] = jnp.concatenate(zcols, axis=1)


def _conv2_kernel(q1_ref, q3_ref, y_ref, dc_ref, b2c_ref, out_ref):
    s0 = jnp.sum(q1_ref[...] * y_ref[0:1, :], axis=1, keepdims=True)
    s1 = jnp.sum(q3_ref[...] * y_ref[1:2, :], axis=1, keepdims=True)
    out_ref[...] = dc_ref[:, 0:1] * s0 + dc_ref[:, 1:2] * s1 + b2c_ref[0, 0]


def kernel(x, ei1, ei2, ei3, ew1, ew2, ew3,
           w1_v1, b1_v1, w2_v1, b2_v1,
           w1_v2, b1_v2, w2_v2, b2_v2,
           w1_v3, b1_v3, w2_v3, b2_v3, att):
    N, F = x.shape
    H = w1_v1.shape[0]

    r1 = _view_full(ei1, ew1, N)
    r2 = _view_full(ei2, ew2, N)
    r3 = _view_full(ei3, ew3, N)

    TD = _tile(N, 512)
    deg_call = pl.pallas_call(
        _deg_kernel,
        out_shape=jax.ShapeDtypeStruct((N, 1), jnp.float32),
        grid=(N // TD,),
        in_specs=[pl.BlockSpec((TD, N), lambda i: (i, 0))],
        out_specs=pl.BlockSpec((TD, 1), lambda i: (i, 0)),
        compiler_params=pltpu.CompilerParams(
            dimension_semantics=("parallel",)),
    )
    q1 = _view_full(ei1, None, N)
    q3 = _view_full(ei3, None, N)
    di1, di2, di3, qd1, qd3 = [
        jnp.where(d > 0.0, jax.lax.rsqrt(d), 0.0).reshape(N)
        for d in (deg_call(r1), deg_call(r2), deg_call(r3),
                  deg_call(q1), deg_call(q3))]

    drow = jnp.stack([di1.reshape(1, N), di2.reshape(1, N),
                      di3.reshape(1, N)])
    dcol = jnp.stack([di1.reshape(N, 1), di2.reshape(N, 1),
                      di3.reshape(N, 1)])

    w1_all = jnp.concatenate(
        [w1_v1.T, w1_v2.T, w1_v3.T], axis=1).astype(jnp.bfloat16)
    b1_all = jnp.stack([b1_v1.reshape(1, H).astype(jnp.float32),
                        b1_v2.reshape(1, H).astype(jnp.float32),
                        b1_v3.reshape(1, H).astype(jnp.float32)])

    s = jax.nn.softmax(att.reshape(3).astype(jnp.float32))
    c0, c2 = s[0], s[1] + s[2]
    w2_rows = jnp.stack([c0 * w2_v1.reshape(1, H).astype(jnp.float32),
                         c2 * w2_v3.reshape(1, H).astype(jnp.float32)])
    b2c = (c0 * b2_v1.reshape(()).astype(jnp.float32)
           + c2 * b2_v3.reshape(()).astype(jnp.float32)).reshape(1, 1)

    TX = _tile(N, 512)
    xw = pl.pallas_call(
        _xw_kernel,
        out_shape=jax.ShapeDtypeStruct((N, 3 * H), jnp.bfloat16),
        grid=(N // TX,),
        in_specs=[pl.BlockSpec((TX, F), lambda i: (i, 0)),
                  pl.BlockSpec((F, 3 * H), lambda i: (0, 0))],
        out_specs=pl.BlockSpec((TX, 3 * H), lambda i: (i, 0)),
        compiler_params=pltpu.CompilerParams(
            dimension_semantics=("parallel",)),
    )(x, w1_all)

    T = _tile(N, 256)
    features, z = pl.pallas_call(
        functools.partial(_conv1_kernel, nhid=H),
        out_shape=(jax.ShapeDtypeStruct((N, H), x.dtype),
                   jax.ShapeDtypeStruct((N, 2), jnp.float32)),
        grid=(N // T,),
        in_specs=[pl.BlockSpec((T, N), lambda i: (i, 0)),
                  pl.BlockSpec((T, N), lambda i: (i, 0)),
                  pl.BlockSpec((T, N), lambda i: (i, 0)),
                  pl.BlockSpec((N, 3 * H), lambda i: (0, 0)),
                  pl.BlockSpec((3, 1, N), lambda i: (0, 0, 0)),
                  pl.BlockSpec((3, T, 1), lambda i: (0, i, 0)),
                  pl.BlockSpec((3, 1, H), lambda i: (0, 0, 0)),
                  pl.BlockSpec((2, 1, H), lambda i: (0, 0, 0))],
        out_specs=[pl.BlockSpec((T, H), lambda i: (i, 0)),
                   pl.BlockSpec((T, 2), lambda i: (i, 0))],
        compiler_params=pltpu.CompilerParams(
            dimension_semantics=("parallel",)),
    )(r1, r2, r3, xw, drow, dcol, b1_all, w2_rows)

    y = jnp.stack([qd1 * z[:, 0], qd3 * z[:, 1]])
    dc2 = jnp.stack([qd1, qd3], axis=1)
    TC = _tile(N, 256)
    comb = pl.pallas_call(
        _conv2_kernel,
        out_shape=jax.ShapeDtypeStruct((N, 1), x.dtype),
        grid=(N // TC,),
        in_specs=[pl.BlockSpec((TC, N), lambda i: (i, 0)),
                  pl.BlockSpec((TC, N), lambda i: (i, 0)),
                  pl.BlockSpec((2, N), lambda i: (0, 0)),
                  pl.BlockSpec((TC, 2), lambda i: (i, 0)),
                  pl.BlockSpec((1, 1), lambda i: (0, 0))],
        out_specs=pl.BlockSpec((TC, 1), lambda i: (i, 0)),
        compiler_params=pltpu.CompilerParams(
            dimension_semantics=("parallel",)),
    )(q1, q3, y, dc2, b2c)

    return comb.reshape(-1), features
